# baseline (speedup 1.0000x reference)
import functools

import jax
import jax.numpy as jnp
from jax import lax
from jax.experimental import pallas as pl
from jax.experimental.pallas import tpu as pltpu
from jax.experimental.pallas import tpu_sc as plsc

N_EDGES = 1600000
N_BONDS = 800000
N_FRAMES = 800000

NW = 32
EDGES_PER_W = N_EDGES // NW
NORM_Q = 10000
NORM_NCHUNK = EDGES_PER_W // NORM_Q

HALF_FRAMES = N_FRAMES // 2
CHUNK_F = 6272
CHUNK_E = 2 * CHUNK_F
NCHUNK = 4
G_BIG = 1563
G_SMALL = 1562

_MESH = dict(core_axis_name="c", subcore_axis_name="s")


def _newton_norm(s):
    s = jnp.maximum(s, jnp.float32(1e-20))
    bits = plsc.bitcast(s, jnp.int32)
    r = plsc.bitcast(jnp.int32(0x5F3759DF) - (bits >> 1), jnp.float32)
    r = r * (1.5 - 0.5 * s * r * r)
    r = r * (1.5 - 0.5 * s * r * r)
    r = r * (1.5 - 0.5 * s * r * r)
    return s * r


def _norms_sc(rx, ry, rz):
    @functools.partial(
        pl.kernel,
        mesh=plsc.VectorSubcoreMesh(**_MESH),
        compiler_params=pltpu.CompilerParams(needs_layout_passes=False),
        out_type=jax.ShapeDtypeStruct((N_EDGES,), jnp.float32),
        scratch_types=[
            pltpu.VMEM((NORM_Q,), jnp.float32),
            pltpu.VMEM((NORM_Q,), jnp.float32),
            pltpu.VMEM((NORM_Q,), jnp.float32),
            pltpu.VMEM((NORM_Q,), jnp.float32),
            pltpu.VMEM((NORM_Q,), jnp.float32),
            pltpu.VMEM((NORM_Q,), jnp.float32),
            pltpu.VMEM((NORM_Q,), jnp.float32),
            pltpu.SemaphoreType.DMA,
            pltpu.SemaphoreType.DMA,
        ],
    )
    def body(x_hbm, y_hbm, z_hbm, d_hbm,
             x0, y0, z0, x1, y1, z1, d_v, sem0, sem1):
        wid = lax.axis_index("c") * 16 + lax.axis_index("s")
        start_e = wid * EDGES_PER_W
        bufs = ((x0, y0, z0, sem0), (x1, y1, z1, sem1))

        def start_in(c):
            xb, yb, zb, sem = bufs[c % 2]
            base_e = start_e + c * NORM_Q
            hx = pltpu.async_copy(x_hbm.at[pl.ds(base_e, NORM_Q)], xb, sem)
            hy = pltpu.async_copy(y_hbm.at[pl.ds(base_e, NORM_Q)], yb, sem)
            hz = pltpu.async_copy(z_hbm.at[pl.ds(base_e, NORM_Q)], zb, sem)
            return (hx, hy, hz)

        handles = start_in(0)
        for c in range(NORM_NCHUNK):
            for h in handles:
                h.wait()
            if c + 1 < NORM_NCHUNK:
                handles = start_in(c + 1)
            xb, yb, zb, _ = bufs[c % 2]

            def vec_body(g, carry2, xb=xb, yb=yb, zb=zb):
                b = g * 16
                x = xb[pl.ds(b, 16)]
                y = yb[pl.ds(b, 16)]
                z = zb[pl.ds(b, 16)]
                d_v[pl.ds(b, 16)] = _newton_norm(x * x + y * y + z * z)
                return carry2

            lax.fori_loop(0, NORM_Q // 16, vec_body, 0)
            pltpu.sync_copy(d_v, d_hbm.at[pl.ds(start_e + c * NORM_Q,
                                                NORM_Q)])

    return body(rx, ry, rz)


def _energy_sc(d, idx_of_bonds, bond_types, stiffness, equilibrium):
    @functools.partial(
        pl.kernel,
        mesh=plsc.VectorSubcoreMesh(**_MESH),
        compiler_params=pltpu.CompilerParams(needs_layout_passes=False),
        out_type=jax.ShapeDtypeStruct((N_FRAMES,), jnp.float32),
        scratch_types=[
            pltpu.VMEM((CHUNK_E,), jnp.int32),
            pltpu.VMEM((CHUNK_E,), jnp.int32),
            pltpu.VMEM((CHUNK_E,), jnp.float32),
            pltpu.VMEM((CHUNK_E,), jnp.float32),
            pltpu.VMEM((CHUNK_E,), jnp.int32),
            pltpu.VMEM((CHUNK_E,), jnp.int32),
            pltpu.VMEM((CHUNK_E,), jnp.float32),
            pltpu.VMEM((CHUNK_F,), jnp.float32),
            pltpu.VMEM((16,), jnp.float32),
            pltpu.VMEM((16,), jnp.float32),
            pltpu.SemaphoreType.DMA,
            pltpu.SemaphoreType.DMA,
        ],
    )
    def body(d_hbm, idx_hbm, bt_hbm, stiff_hbm, eq_hbm, out_hbm,
             idx0, idx1, g0, g1, bt0, bt1, e_v, o_v, k_v, r0_v,
             semg0, semg1):
        half = lax.axis_index("c")
        pos = lax.axis_index("s")
        groups = jnp.where(pos < 8, G_BIG, G_SMALL)
        start_g = pos * G_SMALL + jnp.minimum(pos, 8)
        start_f = half * HALF_FRAMES + 16 * start_g
        end_f = start_f + 16 * groups

        pltpu.sync_copy(stiff_hbm, k_v)
        pltpu.sync_copy(eq_hbm, r0_v)

        iota = lax.iota(jnp.int32, 16)
        idx_b = (idx0, idx1)
        g_b = (g0, g1)
        bt_b = (bt0, bt1)
        sem_b = (semg0, semg1)

        bases = []
        for c in range(NCHUNK):
            bases.append(jnp.minimum(start_f + c * CHUNK_F, end_f - CHUNK_F))

        def load_ib(c):
            base_e = 2 * bases[c]
            bt_off = base_e - half * N_BONDS
            pltpu.sync_copy(idx_hbm.at[pl.ds(base_e, CHUNK_E)], idx_b[c % 2])
            pltpu.sync_copy(bt_hbm.at[pl.ds(bt_off, CHUNK_E)], bt_b[c % 2])

        def start_gather(c):
            return pltpu.async_copy(d_hbm.at[idx_b[c % 2]], g_b[c % 2],
                                    sem_b[c % 2])

        load_ib(0)
        gh = {0: start_gather(0)}
        load_ib(1)

        for c in range(NCHUNK):
            gh.pop(c).wait()
            if c + 1 < NCHUNK:
                gh[c + 1] = start_gather(c + 1)
            g_v = g_b[c % 2]
            t_v = bt_b[c % 2]

            def vec_body(g, carry2, g_v=g_v, t_v=t_v):
                b = g * 16
                dist = _newton_norm(g_v[pl.ds(b, 16)])
                t = t_v[pl.ds(b, 16)]
                k = plsc.load_gather(k_v, [t])
                r0 = plsc.load_gather(r0_v, [t])
                dd = dist - r0
                e_v[pl.ds(b, 16)] = k * dd * dd
                return carry2

            lax.fori_loop(0, CHUNK_E // 16, vec_body, 0)

            def pair_body(h, carry2):
                b = h * 16
                j = 2 * (b + iota)
                ev = plsc.load_gather(e_v, [j])
                ov = plsc.load_gather(e_v, [j + 1])
                o_v[pl.ds(b, 16)] = 0.5 * (ev + ov)
                return carry2

            lax.fori_loop(0, CHUNK_F // 16, pair_body, 0)
            pltpu.sync_copy(o_v, out_hbm.at[pl.ds(bases[c], CHUNK_F)])
            if c + 2 < NCHUNK:
                load_ib(c + 2)

    return body(d, idx_of_bonds, bond_types, stiffness, equilibrium)


def kernel(Rij, idx_i, idx_of_bonds, bonds_list, bond_types, n_bonds,
           stiffness, equilibrium_value):
    rt = jnp.transpose(Rij)
    s = jnp.sum(rt * rt, axis=0)
    return _energy_sc(s, idx_of_bonds, bond_types, stiffness,
                      equilibrium_value)


# speedup vs baseline: 1.0154x; 1.0154x over previous
import functools

import jax
import jax.numpy as jnp
from jax import lax
from jax.experimental import pallas as pl
from jax.experimental.pallas import tpu as pltpu
from jax.experimental.pallas import tpu_sc as plsc

N_EDGES = 1600000
N_BONDS = 800000
N_FRAMES = 800000

NW = 32
EDGES_PER_W = N_EDGES // NW
NORM_Q = 10000
NORM_NCHUNK = EDGES_PER_W // NORM_Q

HALF_FRAMES = N_FRAMES // 2
CHUNK_F = 3136
CHUNK_E = 2 * CHUNK_F
NCHUNK = 8
G_BIG = 1563
G_SMALL = 1562

_MESH = dict(core_axis_name="c", subcore_axis_name="s")


def _newton_norm(s):
    s = jnp.maximum(s, jnp.float32(1e-20))
    bits = plsc.bitcast(s, jnp.int32)
    r = plsc.bitcast(jnp.int32(0x5F3759DF) - (bits >> 1), jnp.float32)
    r = r * (1.5 - 0.5 * s * r * r)
    r = r * (1.5 - 0.5 * s * r * r)
    return s * r


def _norms_sc(rx, ry, rz):
    @functools.partial(
        pl.kernel,
        mesh=plsc.VectorSubcoreMesh(**_MESH),
        compiler_params=pltpu.CompilerParams(needs_layout_passes=False),
        out_type=jax.ShapeDtypeStruct((N_EDGES,), jnp.float32),
        scratch_types=[
            pltpu.VMEM((NORM_Q,), jnp.float32),
            pltpu.VMEM((NORM_Q,), jnp.float32),
            pltpu.VMEM((NORM_Q,), jnp.float32),
            pltpu.VMEM((NORM_Q,), jnp.float32),
            pltpu.VMEM((NORM_Q,), jnp.float32),
            pltpu.VMEM((NORM_Q,), jnp.float32),
            pltpu.VMEM((NORM_Q,), jnp.float32),
            pltpu.SemaphoreType.DMA,
            pltpu.SemaphoreType.DMA,
        ],
    )
    def body(x_hbm, y_hbm, z_hbm, d_hbm,
             x0, y0, z0, x1, y1, z1, d_v, sem0, sem1):
        wid = lax.axis_index("c") * 16 + lax.axis_index("s")
        start_e = wid * EDGES_PER_W
        bufs = ((x0, y0, z0, sem0), (x1, y1, z1, sem1))

        def start_in(c):
            xb, yb, zb, sem = bufs[c % 2]
            base_e = start_e + c * NORM_Q
            hx = pltpu.async_copy(x_hbm.at[pl.ds(base_e, NORM_Q)], xb, sem)
            hy = pltpu.async_copy(y_hbm.at[pl.ds(base_e, NORM_Q)], yb, sem)
            hz = pltpu.async_copy(z_hbm.at[pl.ds(base_e, NORM_Q)], zb, sem)
            return (hx, hy, hz)

        handles = start_in(0)
        for c in range(NORM_NCHUNK):
            for h in handles:
                h.wait()
            if c + 1 < NORM_NCHUNK:
                handles = start_in(c + 1)
            xb, yb, zb, _ = bufs[c % 2]

            def vec_body(g, carry2, xb=xb, yb=yb, zb=zb):
                b = g * 16
                x = xb[pl.ds(b, 16)]
                y = yb[pl.ds(b, 16)]
                z = zb[pl.ds(b, 16)]
                d_v[pl.ds(b, 16)] = _newton_norm(x * x + y * y + z * z)
                return carry2

            lax.fori_loop(0, NORM_Q // 16, vec_body, 0)
            pltpu.sync_copy(d_v, d_hbm.at[pl.ds(start_e + c * NORM_Q,
                                                NORM_Q)])

    return body(rx, ry, rz)


def _energy_sc(d, idx_of_bonds, bond_types, stiffness, equilibrium):
    @functools.partial(
        pl.kernel,
        mesh=plsc.VectorSubcoreMesh(**_MESH),
        compiler_params=pltpu.CompilerParams(needs_layout_passes=False),
        out_type=jax.ShapeDtypeStruct((N_FRAMES,), jnp.float32),
        scratch_types=[
            pltpu.VMEM((CHUNK_E,), jnp.int32),
            pltpu.VMEM((CHUNK_E,), jnp.int32),
            pltpu.VMEM((CHUNK_E,), jnp.float32),
            pltpu.VMEM((CHUNK_E,), jnp.float32),
            pltpu.VMEM((CHUNK_E,), jnp.int32),
            pltpu.VMEM((CHUNK_E,), jnp.int32),
            pltpu.VMEM((CHUNK_E,), jnp.float32),
            pltpu.VMEM((CHUNK_F,), jnp.float32),
            pltpu.VMEM((16,), jnp.float32),
            pltpu.VMEM((16,), jnp.float32),
            pltpu.SemaphoreType.DMA,
            pltpu.SemaphoreType.DMA,
        ],
    )
    def body(d_hbm, idx_hbm, bt_hbm, stiff_hbm, eq_hbm, out_hbm,
             idx0, idx1, g0, g1, bt0, bt1, e_v, o_v, k_v, r0_v,
             semg0, semg1):
        half = lax.axis_index("c")
        pos = lax.axis_index("s")
        groups = jnp.where(pos < 8, G_BIG, G_SMALL)
        start_g = pos * G_SMALL + jnp.minimum(pos, 8)
        start_f = half * HALF_FRAMES + 16 * start_g
        end_f = start_f + 16 * groups

        pltpu.sync_copy(stiff_hbm, k_v)
        pltpu.sync_copy(eq_hbm, r0_v)
        k_v[...] = 0.5 * k_v[...]

        iota = lax.iota(jnp.int32, 16)
        evens = 2 * iota
        odds = evens + 1
        idx_b = (idx0, idx1)
        g_b = (g0, g1)
        bt_b = (bt0, bt1)
        sem_b = (semg0, semg1)

        bases = []
        for c in range(NCHUNK):
            bases.append(jnp.minimum(start_f + c * CHUNK_F, end_f - CHUNK_F))

        def load_ib(c):
            base_e = 2 * bases[c]
            bt_off = base_e - half * N_BONDS
            pltpu.sync_copy(idx_hbm.at[pl.ds(base_e, CHUNK_E)], idx_b[c % 2])
            pltpu.sync_copy(bt_hbm.at[pl.ds(bt_off, CHUNK_E)], bt_b[c % 2])

        def start_gather(c):
            return pltpu.async_copy(d_hbm.at[idx_b[c % 2]], g_b[c % 2],
                                    sem_b[c % 2])

        load_ib(0)
        gh = {0: start_gather(0)}
        load_ib(1)

        for c in range(NCHUNK):
            gh.pop(c).wait()
            if c + 1 < NCHUNK:
                gh[c + 1] = start_gather(c + 1)
            g_v = g_b[c % 2]
            t_v = bt_b[c % 2]

            def grp_body(gq, carry2, g_v=g_v, t_v=t_v):
                b32 = gq * 32
                d0 = _newton_norm(g_v[pl.ds(b32, 16)])
                d1 = _newton_norm(g_v[pl.ds(b32 + 16, 16)])
                t0 = t_v[pl.ds(b32, 16)]
                t1 = t_v[pl.ds(b32 + 16, 16)]
                k0 = plsc.load_gather(k_v, [t0])
                k1 = plsc.load_gather(k_v, [t1])
                r00 = plsc.load_gather(r0_v, [t0])
                r01 = plsc.load_gather(r0_v, [t1])
                dd0 = d0 - r00
                dd1 = d1 - r01
                e_v[pl.ds(b32, 16)] = k0 * dd0 * dd0
                e_v[pl.ds(b32 + 16, 16)] = k1 * dd1 * dd1
                ev = plsc.load_gather(e_v, [b32 + evens])
                ov = plsc.load_gather(e_v, [b32 + odds])
                o_v[pl.ds(gq * 16, 16)] = ev + ov
                return carry2

            lax.fori_loop(0, CHUNK_F // 16, grp_body, 0)
            pltpu.sync_copy(o_v, out_hbm.at[pl.ds(bases[c], CHUNK_F)])
            if c + 2 < NCHUNK:
                load_ib(c + 2)

    return body(d, idx_of_bonds, bond_types, stiffness, equilibrium)


def kernel(Rij, idx_i, idx_of_bonds, bonds_list, bond_types, n_bonds,
           stiffness, equilibrium_value):
    s = jnp.sum(Rij * Rij, axis=1)
    return _energy_sc(s, idx_of_bonds, bond_types, stiffness,
                      equilibrium_value)


# speedup vs baseline: 1.0182x; 1.0028x over previous
import functools

import jax
import jax.numpy as jnp
from jax import lax
from jax.experimental import pallas as pl
from jax.experimental.pallas import tpu as pltpu
from jax.experimental.pallas import tpu_sc as plsc

N_EDGES = 1600000
N_BONDS = 800000
N_FRAMES = 800000

NW = 32
EDGES_PER_W = N_EDGES // NW
NORM_Q = 10000
NORM_NCHUNK = EDGES_PER_W // NORM_Q

HALF_FRAMES = N_FRAMES // 2
CHUNK_F = 3136
CHUNK_E = 2 * CHUNK_F
NCHUNK = 8
G_BIG = 1563
G_SMALL = 1562

_MESH = dict(core_axis_name="c", subcore_axis_name="s")


def _newton_norm(s):
    s = jnp.maximum(s, jnp.float32(1e-20))
    bits = plsc.bitcast(s, jnp.int32)
    r = plsc.bitcast(jnp.int32(0x5F3759DF) - (bits >> 1), jnp.float32)
    r = r * (1.5 - 0.5 * s * r * r)
    r = r * (1.5 - 0.5 * s * r * r)
    return s * r


def _norms_sc(rx, ry, rz):
    @functools.partial(
        pl.kernel,
        mesh=plsc.VectorSubcoreMesh(**_MESH),
        compiler_params=pltpu.CompilerParams(needs_layout_passes=False),
        out_type=jax.ShapeDtypeStruct((N_EDGES,), jnp.float32),
        scratch_types=[
            pltpu.VMEM((NORM_Q,), jnp.float32),
            pltpu.VMEM((NORM_Q,), jnp.float32),
            pltpu.VMEM((NORM_Q,), jnp.float32),
            pltpu.VMEM((NORM_Q,), jnp.float32),
            pltpu.VMEM((NORM_Q,), jnp.float32),
            pltpu.VMEM((NORM_Q,), jnp.float32),
            pltpu.VMEM((NORM_Q,), jnp.float32),
            pltpu.SemaphoreType.DMA,
            pltpu.SemaphoreType.DMA,
        ],
    )
    def body(x_hbm, y_hbm, z_hbm, d_hbm,
             x0, y0, z0, x1, y1, z1, d_v, sem0, sem1):
        wid = lax.axis_index("c") * 16 + lax.axis_index("s")
        start_e = wid * EDGES_PER_W
        bufs = ((x0, y0, z0, sem0), (x1, y1, z1, sem1))

        def start_in(c):
            xb, yb, zb, sem = bufs[c % 2]
            base_e = start_e + c * NORM_Q
            hx = pltpu.async_copy(x_hbm.at[pl.ds(base_e, NORM_Q)], xb, sem)
            hy = pltpu.async_copy(y_hbm.at[pl.ds(base_e, NORM_Q)], yb, sem)
            hz = pltpu.async_copy(z_hbm.at[pl.ds(base_e, NORM_Q)], zb, sem)
            return (hx, hy, hz)

        handles = start_in(0)
        for c in range(NORM_NCHUNK):
            for h in handles:
                h.wait()
            if c + 1 < NORM_NCHUNK:
                handles = start_in(c + 1)
            xb, yb, zb, _ = bufs[c % 2]

            def vec_body(g, carry2, xb=xb, yb=yb, zb=zb):
                b = g * 16
                x = xb[pl.ds(b, 16)]
                y = yb[pl.ds(b, 16)]
                z = zb[pl.ds(b, 16)]
                d_v[pl.ds(b, 16)] = _newton_norm(x * x + y * y + z * z)
                return carry2

            lax.fori_loop(0, NORM_Q // 16, vec_body, 0)
            pltpu.sync_copy(d_v, d_hbm.at[pl.ds(start_e + c * NORM_Q,
                                                NORM_Q)])

    return body(rx, ry, rz)


def _energy_sc(d, idx_of_bonds, bond_types, stiffness, equilibrium):
    @functools.partial(
        pl.kernel,
        mesh=plsc.VectorSubcoreMesh(**_MESH),
        compiler_params=pltpu.CompilerParams(needs_layout_passes=False),
        out_type=jax.ShapeDtypeStruct((N_FRAMES,), jnp.float32),
        scratch_types=[
            pltpu.VMEM((CHUNK_E,), jnp.int32),
            pltpu.VMEM((CHUNK_E,), jnp.int32),
            pltpu.VMEM((CHUNK_E,), jnp.float32),
            pltpu.VMEM((CHUNK_E,), jnp.float32),
            pltpu.VMEM((CHUNK_E,), jnp.int32),
            pltpu.VMEM((CHUNK_E,), jnp.int32),
            pltpu.VMEM((CHUNK_E,), jnp.float32),
            pltpu.VMEM((CHUNK_F,), jnp.float32),
            pltpu.VMEM((16,), jnp.float32),
            pltpu.VMEM((16,), jnp.float32),
            pltpu.SemaphoreType.DMA,
            pltpu.SemaphoreType.DMA,
        ],
    )
    def body(d_hbm, idx_hbm, bt_hbm, stiff_hbm, eq_hbm, out_hbm,
             idx0, idx1, g0, g1, bt0, bt1, e_v, o_v, k_v, r0_v,
             semg0, semg1):
        half = lax.axis_index("c")
        pos = lax.axis_index("s")
        groups = jnp.where(pos < 8, G_BIG, G_SMALL)
        start_g = pos * G_SMALL + jnp.minimum(pos, 8)
        start_f = half * HALF_FRAMES + 16 * start_g
        end_f = start_f + 16 * groups

        pltpu.sync_copy(stiff_hbm, k_v)
        pltpu.sync_copy(eq_hbm, r0_v)
        k_v[...] = 0.5 * k_v[...]

        iota = lax.iota(jnp.int32, 16)
        evens = 2 * iota
        odds = evens + 1
        idx_b = (idx0, idx1)
        g_b = (g0, g1)
        bt_b = (bt0, bt1)
        sem_b = (semg0, semg1)

        bases = []
        for c in range(NCHUNK):
            bases.append(jnp.minimum(start_f + c * CHUNK_F, end_f - CHUNK_F))

        def load_ib(c):
            base_e = 2 * bases[c]
            bt_off = base_e - half * N_BONDS
            pltpu.sync_copy(idx_hbm.at[pl.ds(base_e, CHUNK_E)], idx_b[c % 2])
            pltpu.sync_copy(bt_hbm.at[pl.ds(bt_off, CHUNK_E)], bt_b[c % 2])

        def start_gather(c):
            return pltpu.async_copy(d_hbm.at[idx_b[c % 2]], g_b[c % 2],
                                    sem_b[c % 2])

        load_ib(0)
        gh = {0: start_gather(0)}
        load_ib(1)

        for c in range(NCHUNK):
            gh.pop(c).wait()
            if c + 1 < NCHUNK:
                gh[c + 1] = start_gather(c + 1)
            g_v = g_b[c % 2]
            t_v = bt_b[c % 2]

            def grp_body(gq, carry2, g_v=g_v, t_v=t_v):
                b32 = gq * 32
                d0 = _newton_norm(g_v[pl.ds(b32, 16)])
                d1 = _newton_norm(g_v[pl.ds(b32 + 16, 16)])
                t0 = t_v[pl.ds(b32, 16)]
                t1 = t_v[pl.ds(b32 + 16, 16)]
                k0 = plsc.load_gather(k_v, [t0])
                k1 = plsc.load_gather(k_v, [t1])
                r00 = plsc.load_gather(r0_v, [t0])
                r01 = plsc.load_gather(r0_v, [t1])
                dd0 = d0 - r00
                dd1 = d1 - r01
                e_v[pl.ds(b32, 16)] = k0 * dd0 * dd0
                e_v[pl.ds(b32 + 16, 16)] = k1 * dd1 * dd1
                ev = plsc.load_gather(e_v, [b32 + evens])
                ov = plsc.load_gather(e_v, [b32 + odds])
                o_v[pl.ds(gq * 16, 16)] = ev + ov
                return carry2

            lax.fori_loop(0, CHUNK_F // 16, grp_body, 0)
            pltpu.sync_copy(o_v, out_hbm.at[pl.ds(bases[c], CHUNK_F)])
            if c + 2 < NCHUNK:
                load_ib(c + 2)

    return body(d, idx_of_bonds, bond_types, stiffness, equilibrium)


def kernel(Rij, idx_i, idx_of_bonds, bonds_list, bond_types, n_bonds,
           stiffness, equilibrium_value):
    s = Rij[:, 0] ** 2 + Rij[:, 1] ** 2 + Rij[:, 2] ** 2
    return _energy_sc(s, idx_of_bonds, bond_types, stiffness,
                      equilibrium_value)


# speedup vs baseline: 1.0191x; 1.0009x over previous
import functools

import jax
import jax.numpy as jnp
from jax import lax
from jax.experimental import pallas as pl
from jax.experimental.pallas import tpu as pltpu
from jax.experimental.pallas import tpu_sc as plsc

N_EDGES = 1600000
N_BONDS = 800000
N_FRAMES = 800000

HALF_FRAMES = N_FRAMES // 2
CHUNK_F = 3136
CHUNK_E = 2 * CHUNK_F
NCHUNK = 8
G_BIG = 1563
G_SMALL = 1562

_MESH = dict(core_axis_name="c", subcore_axis_name="s")


def _newton_norm(s):
    s = jnp.maximum(s, jnp.float32(1e-20))
    bits = plsc.bitcast(s, jnp.int32)
    r = plsc.bitcast(jnp.int32(0x5F3759DF) - (bits >> 1), jnp.float32)
    r = r * (1.5 - 0.5 * s * r * r)
    r = r * (1.5 - 0.5 * s * r * r)
    return s * r


def _energy_sc(s, idx_of_bonds, bond_types, stiffness, equilibrium):
    @functools.partial(
        pl.kernel,
        mesh=plsc.VectorSubcoreMesh(**_MESH),
        compiler_params=pltpu.CompilerParams(needs_layout_passes=False),
        out_type=jax.ShapeDtypeStruct((N_FRAMES,), jnp.float32),
        scratch_types=[
            pltpu.VMEM((CHUNK_E,), jnp.int32),
            pltpu.VMEM((CHUNK_E,), jnp.int32),
            pltpu.VMEM((CHUNK_E,), jnp.float32),
            pltpu.VMEM((CHUNK_E,), jnp.float32),
            pltpu.VMEM((CHUNK_E,), jnp.int32),
            pltpu.VMEM((CHUNK_E,), jnp.int32),
            pltpu.VMEM((CHUNK_E,), jnp.float32),
            pltpu.VMEM((CHUNK_F,), jnp.float32),
            pltpu.VMEM((16,), jnp.float32),
            pltpu.VMEM((16,), jnp.float32),
            pltpu.SemaphoreType.DMA,
            pltpu.SemaphoreType.DMA,
        ],
    )
    def body(s_hbm, idx_hbm, bt_hbm, stiff_hbm, eq_hbm, out_hbm,
             idx0, idx1, g0, g1, bt0, bt1, e_v, o_v, k_v, r0_v,
             semg0, semg1):
        half = lax.axis_index("c")
        pos = lax.axis_index("s")
        groups = jnp.where(pos < 8, G_BIG, G_SMALL)
        start_g = pos * G_SMALL + jnp.minimum(pos, 8)
        start_f = half * HALF_FRAMES + 16 * start_g
        end_f = start_f + 16 * groups

        pltpu.sync_copy(stiff_hbm, k_v)
        pltpu.sync_copy(eq_hbm, r0_v)
        k_v[...] = 0.5 * k_v[...]

        iota = lax.iota(jnp.int32, 16)
        evens = 2 * iota
        odds = evens + 1
        idx_b = (idx0, idx1)
        g_b = (g0, g1)
        bt_b = (bt0, bt1)
        sem_b = (semg0, semg1)

        bases = []
        for c in range(NCHUNK):
            bases.append(jnp.minimum(start_f + c * CHUNK_F, end_f - CHUNK_F))

        def load_ib(c):
            base_e = 2 * bases[c]
            bt_off = base_e - half * N_BONDS
            pltpu.sync_copy(idx_hbm.at[pl.ds(base_e, CHUNK_E)], idx_b[c % 2])
            pltpu.sync_copy(bt_hbm.at[pl.ds(bt_off, CHUNK_E)], bt_b[c % 2])

        def start_gather(c):
            return pltpu.async_copy(s_hbm.at[idx_b[c % 2]], g_b[c % 2],
                                    sem_b[c % 2])

        load_ib(0)
        gh = {0: start_gather(0)}
        load_ib(1)

        for c in range(NCHUNK):
            gh.pop(c).wait()
            if c + 1 < NCHUNK:
                gh[c + 1] = start_gather(c + 1)
            g_v = g_b[c % 2]
            t_v = bt_b[c % 2]

            def grp_body(gq, carry2, g_v=g_v, t_v=t_v):
                b32 = gq * 32
                d0 = _newton_norm(g_v[pl.ds(b32, 16)])
                d1 = _newton_norm(g_v[pl.ds(b32 + 16, 16)])
                t0 = t_v[pl.ds(b32, 16)]
                t1 = t_v[pl.ds(b32 + 16, 16)]
                k0 = plsc.load_gather(k_v, [t0])
                k1 = plsc.load_gather(k_v, [t1])
                r00 = plsc.load_gather(r0_v, [t0])
                r01 = plsc.load_gather(r0_v, [t1])
                dd0 = d0 - r00
                dd1 = d1 - r01
                e_v[pl.ds(b32, 16)] = k0 * dd0 * dd0
                e_v[pl.ds(b32 + 16, 16)] = k1 * dd1 * dd1
                ev = plsc.load_gather(e_v, [b32 + evens])
                ov = plsc.load_gather(e_v, [b32 + odds])
                o_v[pl.ds(gq * 16, 16)] = ev + ov
                return carry2

            lax.fori_loop(0, CHUNK_F // 16, grp_body, 0)
            pltpu.sync_copy(o_v, out_hbm.at[pl.ds(bases[c], CHUNK_F)])
            if c + 2 < NCHUNK:
                load_ib(c + 2)

    return body(s, idx_of_bonds, bond_types, stiffness, equilibrium)


def kernel(Rij, idx_i, idx_of_bonds, bonds_list, bond_types, n_bonds,
           stiffness, equilibrium_value):
    s = Rij[:, 0] ** 2 + Rij[:, 1] ** 2 + Rij[:, 2] ** 2
    return _energy_sc(s, idx_of_bonds, bond_types, stiffness,
                      equilibrium_value)
